# Initial kernel scaffold; baseline (speedup 1.0000x reference)
import jax
import jax.numpy as jnp
from jax.experimental import pallas as pl


def kernel(node_features, edge_index, edge_dist, batch, W0, b0, Wf1, bf1, Wf2, bf2, Wl, bl, Wg1, bg1, Wg2, bg2, Wp, bp, ln_g, ln_b):
    raise NotImplementedError("write your pallas kernel here")


# speedup vs baseline: 1.5432x; 1.5432x over previous
import functools

import jax
import jax.numpy as jnp
from jax import lax
from jax.experimental import pallas as pl
from jax.experimental.pallas import tpu as pltpu
from jax.experimental.pallas import tpu_sc as plsc

_N = 50000
_E = 800000
_H = 64
_RBF = 50
_G = 64
_OUT = 64
_NI = 3
_GAMMA = 10.0

_TE = 2000
_TN = 2000
_NHALF = _N // 2
_NPAD = 26624
_CHUNK = 128
_NCH = _E // _CHUNK
_ZROWS = _NPAD // 16
_OCH = 200
_NOCH = _NHALF // _OCH


def _silu(x):
    return x * jax.nn.sigmoid(x)


def _embed_body(x_ref, w_ref, b_ref, o_ref):
    t = jnp.dot(x_ref[...], w_ref[...], preferred_element_type=jnp.float32)
    o_ref[...] = _silu(t + b_ref[...])


def _embed(x, W0, b0):
    return pl.pallas_call(
        _embed_body,
        grid=(_N // _TN,),
        in_specs=[
            pl.BlockSpec((_TN, 3), lambda b: (b, 0)),
            pl.BlockSpec((3, _H), lambda b: (0, 0)),
            pl.BlockSpec((1, _H), lambda b: (0, 0)),
        ],
        out_specs=pl.BlockSpec((_TN, _H), lambda b: (b, 0)),
        out_shape=jax.ShapeDtypeStruct((_N, _H), jnp.float32),
    )(x, W0, b0)


def _filter_body(d_ref, wf1_ref, bf1_ref, wf2_ref, bf2_ref, o0, o1, o2):
    d = d_ref[...]
    centers = lax.broadcasted_iota(jnp.int32, (1, _RBF), 1).astype(jnp.float32) * (
        6.0 / (_RBF - 1))
    diff = d - centers
    rbf = jnp.exp(-_GAMMA * diff * diff)
    outs = (o0, o1, o2)
    for i in range(_NI):
        t = jnp.dot(rbf, wf1_ref[i], preferred_element_type=jnp.float32) + bf1_ref[i]
        t = _silu(t)
        outs[i][...] = (
            jnp.dot(t, wf2_ref[i], preferred_element_type=jnp.float32) + bf2_ref[i]
        )


def _filters(edge_dist, Wf1, bf1, Wf2, bf2):
    return pl.pallas_call(
        _filter_body,
        grid=(_E // _TE,),
        in_specs=[
            pl.BlockSpec((_TE, 1), lambda b: (b, 0)),
            pl.BlockSpec((_NI, _RBF, _H), lambda b: (0, 0, 0)),
            pl.BlockSpec((_NI, _H), lambda b: (0, 0)),
            pl.BlockSpec((_NI, _H, _H), lambda b: (0, 0, 0)),
            pl.BlockSpec((_NI, _H), lambda b: (0, 0)),
        ],
        out_specs=[pl.BlockSpec((_TE, _H), lambda b: (b, 0))] * _NI,
        out_shape=[jax.ShapeDtypeStruct((_E, _H), jnp.float32)] * _NI,
    )(edge_dist.reshape(_E, 1), Wf1, bf1, Wf2, bf2)


@functools.lru_cache(maxsize=1)
def _make_sc_msg():
    mesh = plsc.VectorSubcoreMesh(core_axis_name="c", subcore_axis_name="s")

    @functools.partial(
        pl.kernel,
        out_type=jax.ShapeDtypeStruct((_N, _H), jnp.float32),
        mesh=mesh,
        scratch_types=[
            pltpu.VMEM((_CHUNK,), jnp.int32),
            pltpu.VMEM((_CHUNK,), jnp.int32),
            pltpu.VMEM((_CHUNK,), jnp.int32),
            pltpu.VMEM((_CHUNK, _H), jnp.float32),
            pltpu.VMEM((_CHUNK, _H), jnp.float32),
            pltpu.VMEM_SHARED((_NPAD, _H), jnp.float32),
            pltpu.SemaphoreType.DMA,
        ],
        compiler_params=pltpu.CompilerParams(use_tc_tiling_on_sc=False),
    )
    def _sc_msg(h_hbm, fil_hbm, src_hbm, dst_hbm, agg_hbm,
                src_v, dst_v, dst2_v, rows_v, fil_v, agg_sh, sem):
        c = lax.axis_index("c")
        s = lax.axis_index("s")

        @pl.loop(0, _CHUNK)
        def _zero_rows(r):
            for j in range(_H // 16):
                fil_v[r, pl.ds(j * 16, 16)] = jnp.zeros((16,), jnp.float32)

        @pl.loop(0, _ZROWS // _CHUNK)
        def _zero_spmem(k):
            pltpu.sync_copy(fil_v, agg_sh.at[pl.ds(s * _ZROWS + k * _CHUNK, _CHUNK)])

        plsc.subcore_barrier()

        @pl.loop(s, _NCH, step=16)
        def _chunk(ch):
            base = ch * _CHUNK
            pltpu.sync_copy(src_hbm.at[pl.ds(base, _CHUNK)], src_v)
            pltpu.sync_copy(dst_hbm.at[pl.ds(base, _CHUNK)], dst_v)
            gat = pltpu.async_copy(h_hbm.at[src_v], rows_v, sem)
            pltpu.sync_copy(fil_hbm.at[pl.ds(base, _CHUNK)], fil_v)
            for j in range(_CHUNK // 16):
                d2 = dst_v[pl.ds(j * 16, 16)] - c * _NHALF
                ok = (d2 >= 0) & (d2 < _NHALF)
                dst2_v[pl.ds(j * 16, 16)] = jnp.where(ok, d2, _NHALF)
            gat.wait()

            @pl.loop(0, _CHUNK, unroll=4)
            def _mul(r):
                for j in range(_H // 16):
                    sl = pl.ds(j * 16, 16)
                    rows_v[r, sl] = rows_v[r, sl] * fil_v[r, sl]

            pltpu.sync_copy(rows_v, agg_sh.at[dst2_v], add=True)

        plsc.subcore_barrier()

        @pl.loop(s, _NOCH, step=16)
        def _out(k):
            pltpu.sync_copy(
                agg_sh.at[pl.ds(k * _OCH, _OCH)],
                agg_hbm.at[pl.ds(c * _NHALF + k * _OCH, _OCH)],
            )

    return _sc_msg


def _update_body(h_ref, agg_ref, w_ref, b_ref, o_ref):
    t = jnp.dot(agg_ref[...], w_ref[...], preferred_element_type=jnp.float32)
    o_ref[...] = h_ref[...] + _silu(t + b_ref[...])


def _update(h, agg, Wl, bl):
    return pl.pallas_call(
        _update_body,
        grid=(_N // _TN,),
        in_specs=[
            pl.BlockSpec((_TN, _H), lambda b: (b, 0)),
            pl.BlockSpec((_TN, _H), lambda b: (b, 0)),
            pl.BlockSpec((_H, _H), lambda b: (0, 0)),
            pl.BlockSpec((1, _H), lambda b: (0, 0)),
        ],
        out_specs=pl.BlockSpec((_TN, _H), lambda b: (b, 0)),
        out_shape=jax.ShapeDtypeStruct((_N, _H), jnp.float32),
    )(h, agg, Wl, bl)


def _gate_body(h_ref, b2_ref, wg1_ref, bg1_ref, wg2_ref, bg2_ref, gate_ref, gmax_ref):
    t = jnp.dot(h_ref[...], wg1_ref[...], preferred_element_type=jnp.float32)
    t = _silu(t + bg1_ref[...])
    g = jnp.dot(t, wg2_ref[...], preferred_element_type=jnp.float32) + bg2_ref[...]
    gate_ref[...] = g
    mask = b2_ref[...] == lax.broadcasted_iota(jnp.int32, (_TN, _G), 1)
    tmax = jnp.max(jnp.where(mask, g, -1e30), axis=0, keepdims=True)

    @pl.when(pl.program_id(0) == 0)
    def _():
        gmax_ref[...] = jnp.full((1, _G), -1e30, jnp.float32)

    gmax_ref[...] = jnp.maximum(gmax_ref[...], tmax)


def _gate(h, batch2, Wg1, bg1, Wg2, bg2):
    return pl.pallas_call(
        _gate_body,
        grid=(_N // _TN,),
        in_specs=[
            pl.BlockSpec((_TN, _H), lambda b: (b, 0)),
            pl.BlockSpec((_TN, 1), lambda b: (b, 0)),
            pl.BlockSpec((_H, _H // 2), lambda b: (0, 0)),
            pl.BlockSpec((1, _H // 2), lambda b: (0, 0)),
            pl.BlockSpec((_H // 2, 1), lambda b: (0, 0)),
            pl.BlockSpec((1, 1), lambda b: (0, 0)),
        ],
        out_specs=[
            pl.BlockSpec((_TN, 1), lambda b: (b, 0)),
            pl.BlockSpec((1, _G), lambda b: (0, 0)),
        ],
        out_shape=[
            jax.ShapeDtypeStruct((_N, 1), jnp.float32),
            jax.ShapeDtypeStruct((1, _G), jnp.float32),
        ],
    )(h, batch2, Wg1, bg1, Wg2, bg2)


def _pool_body(h_ref, b2_ref, gate_ref, gmax_ref, num_ref, den_ref):
    mask = b2_ref[...] == lax.broadcasted_iota(jnp.int32, (_TN, _G), 1)
    gmax_n = jnp.max(jnp.where(mask, gmax_ref[...], -1e30), axis=1, keepdims=True)
    e = jnp.exp(gate_ref[...] - gmax_n)
    em = jnp.where(mask, e, 0.0)
    ntile = lax.dot_general(em, h_ref[...], (((0,), (0,)), ((), ())),
                            preferred_element_type=jnp.float32)
    ones = jnp.ones((_TN, 1), jnp.float32)
    dtile = lax.dot_general(em, ones, (((0,), (0,)), ((), ())),
                            preferred_element_type=jnp.float32)

    @pl.when(pl.program_id(0) == 0)
    def _():
        num_ref[...] = jnp.zeros((_G, _H), jnp.float32)
        den_ref[...] = jnp.zeros((_G, 1), jnp.float32)

    num_ref[...] += ntile
    den_ref[...] += dtile


def _pool(h, batch2, gate, gmax):
    return pl.pallas_call(
        _pool_body,
        grid=(_N // _TN,),
        in_specs=[
            pl.BlockSpec((_TN, _H), lambda b: (b, 0)),
            pl.BlockSpec((_TN, 1), lambda b: (b, 0)),
            pl.BlockSpec((_TN, 1), lambda b: (b, 0)),
            pl.BlockSpec((1, _G), lambda b: (0, 0)),
        ],
        out_specs=[
            pl.BlockSpec((_G, _H), lambda b: (0, 0)),
            pl.BlockSpec((_G, 1), lambda b: (0, 0)),
        ],
        out_shape=[
            jax.ShapeDtypeStruct((_G, _H), jnp.float32),
            jax.ShapeDtypeStruct((_G, 1), jnp.float32),
        ],
    )(h, batch2, gate, gmax)


def _final_body(num_ref, den_ref, wp_ref, bp_ref, g_ref, b_ref, o_ref):
    hg = num_ref[...] / (den_ref[...] + 1e-8)
    z = jnp.dot(hg, wp_ref[...], preferred_element_type=jnp.float32) + bp_ref[...]
    mu = jnp.mean(z, axis=-1, keepdims=True)
    var = jnp.mean((z - mu) ** 2, axis=-1, keepdims=True)
    zn = (z - mu) / jnp.sqrt(var + 1e-5) * g_ref[...] + b_ref[...]
    o_ref[...] = jax.nn.gelu(zn)


def _final(num, den, Wp, bp, ln_g, ln_b):
    return pl.pallas_call(
        _final_body,
        in_specs=[pl.BlockSpec(x.shape, lambda: tuple(0 for _ in x.shape))
                  for x in (num, den, Wp, bp, ln_g, ln_b)],
        out_specs=pl.BlockSpec((_G, _OUT), lambda: (0, 0)),
        out_shape=jax.ShapeDtypeStruct((_G, _OUT), jnp.float32),
    )(num, den, Wp, bp, ln_g, ln_b)


def kernel(node_features, edge_index, edge_dist, batch,
           W0, b0, Wf1, bf1, Wf2, bf2, Wl, bl,
           Wg1, bg1, Wg2, bg2, Wp, bp, ln_g, ln_b):
    src = edge_index[0]
    dst = edge_index[1]
    h = _embed(node_features, W0, b0.reshape(1, _H))
    fils = _filters(edge_dist, Wf1, bf1, Wf2, bf2)
    sc_msg = _make_sc_msg()
    for i in range(_NI):
        agg = sc_msg(h, fils[i], src, dst)
        h = _update(h, agg, Wl[i], bl[i].reshape(1, _H))
    batch2 = batch.reshape(_N, 1)
    gate, gmax = _gate(h, batch2, Wg1, bg1.reshape(1, _H // 2),
                       Wg2, bg2.reshape(1, 1))
    num, den = _pool(h, batch2, gate, gmax)
    return _final(num, den, Wp, bp.reshape(1, _OUT),
                  ln_g.reshape(1, _OUT), ln_b.reshape(1, _OUT))
